# baseline (speedup 1.0000x reference)
import functools

import jax
import jax.numpy as jnp
from jax import lax
from jax.experimental import pallas as pl
from jax.experimental.pallas import tpu as pltpu
from jax.experimental.pallas import tpu_sc as plsc

VOCAB = 100000
D = 128
TILE = 128
NT = -(-VOCAB // TILE)
L = 16


def _sc_matvec(weight_flat):
    info = plsc.get_sparse_core_info()
    nw = info.num_cores * info.num_subcores

    mesh = plsc.VectorSubcoreMesh(core_axis_name="c", subcore_axis_name="s")

    @functools.partial(
        pl.kernel,
        mesh=mesh,
        out_type=jax.ShapeDtypeStruct((VOCAB,), jnp.float32),
        scratch_types=[
            pltpu.VMEM((TILE * D,), jnp.float32),
            pltpu.VMEM((TILE,), jnp.float32),
        ],
        compiler_params=pltpu.CompilerParams(needs_layout_passes=False),
    )
    def k(w_hbm, out_hbm, wbuf, obuf):
        wid = lax.axis_index("s") * info.num_cores + lax.axis_index("c")
        lane = lax.iota(jnp.int32, L)
        rowword = lane * D
        n_tiles = (NT - 1 - wid) // nw + 1

        def tile_body(i, carry):
            t = wid + nw * i
            base = jnp.minimum(t * TILE, VOCAB - TILE)
            pltpu.sync_copy(w_hbm.at[pl.ds(base * D, TILE * D)], wbuf)

            def group_body(g, c2):
                idx = g * (L * D) + rowword
                acc = jnp.zeros((L,), jnp.float32)
                for col in range(1, D):
                    idx = idx + 1
                    v = plsc.load_gather(wbuf, [idx])
                    acc = acc + v * float(col)
                obuf[pl.ds(g * L, L)] = acc
                return c2

            lax.fori_loop(0, TILE // L, group_body, 0)
            pltpu.sync_copy(obuf, out_hbm.at[pl.ds(base, TILE)])
            return carry

        lax.fori_loop(0, n_tiles, tile_body, 0)

    return k(weight_flat)


def kernel(x, weight):
    del x
    return _sc_matvec(weight.reshape(-1))
